# baseline (speedup 1.0000x reference)
import functools

import jax
import jax.numpy as jnp
from jax import lax
from jax.experimental import pallas as pl
from jax.experimental.pallas import tpu as pltpu
from jax.experimental.pallas import tpu_sc as plsc

NC = 2
NS = 16
NW = NC * NS
L = 16


def _mm_body(x_ref, wt_ref, b_ref, h_ref):
    h_ref[...] = (
        jnp.dot(x_ref[...], wt_ref[...], preferred_element_type=jnp.float32)
        + b_ref[...]
    )


@functools.partial(jax.jit, static_argnames=())
def _matmul(x, wt, b2d):
    n, d_in = x.shape
    d_out = wt.shape[1]
    blk = 1000 if n % 1000 == 0 else n
    grid = n // blk
    return pl.pallas_call(
        _mm_body,
        grid=(grid,),
        in_specs=[
            pl.BlockSpec((blk, d_in), lambda i: (i, 0)),
            pl.BlockSpec((d_in, d_out), lambda i: (0, 0)),
            pl.BlockSpec((1, d_out), lambda i: (0, 0)),
        ],
        out_specs=pl.BlockSpec((blk, d_out), lambda i: (i, 0)),
        out_shape=jax.ShapeDtypeStruct((n, d_out), jnp.float32),
    )(x, wt, b2d)


def _comb_body(p_ref, o_ref):
    o_ref[...] = jnp.maximum(p_ref[0] + p_ref[1], 0.0)


def _combine(partials, n):
    _, _, d = partials.shape
    blk = 1000 if n % 1000 == 0 else n
    grid = n // blk
    return pl.pallas_call(
        _comb_body,
        grid=(grid,),
        in_specs=[pl.BlockSpec((2, blk, d), lambda i: (0, i, 0))],
        out_specs=pl.BlockSpec((blk, d), lambda i: (i, 0)),
        out_shape=jax.ShapeDtypeStruct((n, d), jnp.float32),
    )(partials)


def _sc_mp(h, packed, e, k, n_chunks):
    n, d = h.shape
    n_pad = -(-n // (NS * 8)) * (NS * 8)
    rows_tile = n_pad // NS
    dk = 32
    spans = []
    off = 0
    while off < rows_tile:
        spans.append((off, min(dk, rows_tile - off)))
        off += dk

    mesh = plsc.VectorSubcoreMesh(
        core_axis_name="c", subcore_axis_name="s",
        num_cores=NC, num_subcores=NS)

    @functools.partial(
        pl.kernel,
        out_type=jax.ShapeDtypeStruct((NC, n_pad, d), jnp.float32),
        mesh=mesh,
        scratch_types=[
            [pltpu.VMEM((3, k), jnp.int32) for _ in range(3)],
            [pltpu.VMEM((k, d), jnp.float32) for _ in range(3)],
            pltpu.VMEM((dk, d), jnp.float32),
            pltpu.VMEM_SHARED((n_pad, d), jnp.float32),
            [pltpu.SemaphoreType.DMA for _ in range(3)],
            [pltpu.SemaphoreType.DMA for _ in range(3)],
            [pltpu.SemaphoreType.DMA for _ in range(3)],
        ],
    )
    def mp(h_hbm, pk_hbm, out_hbm,
           eb, msg, drain_v, acc_sh, sem_i, sem_g, sem_s):
        c = lax.axis_index("c")
        s = lax.axis_index("s")
        wid = c * NS + s

        def zrow(r, _):
            for j in range(d // L):
                drain_v[r, pl.ds(j * L, L)] = jnp.zeros((L,), jnp.float32)
            return 0

        lax.fori_loop(0, dk, zrow, 0)

        for off, sz in spans:
            pltpu.sync_copy(
                drain_v.at[pl.ds(0, sz)],
                acc_sh.at[pl.ds(s * rows_tile + off, sz)])
        plsc.subcore_barrier()

        chunk0 = wid * n_chunks

        def load_idx(ci, m):
            pltpu.async_copy(pk_hbm.at[chunk0 + ci], eb[m], sem_i[m])

        def wait_idx(m):
            pltpu.make_async_copy(pk_hbm.at[0], eb[m], sem_i[m]).wait()

        def start_gather(m):
            pltpu.async_copy(h_hbm.at[eb[m].at[1]], msg[m], sem_g[m])

        def wait_gather(m):
            pltpu.make_async_copy(h_hbm.at[pl.ds(0, k)], msg[m],
                                  sem_g[m]).wait()

        def start_scatter(m):
            pltpu.async_copy(msg[m], acc_sh.at[eb[m].at[0]], sem_s[m],
                             add=True)

        def wait_scatter(m):
            pltpu.make_async_copy(msg[m], acc_sh.at[pl.ds(0, k)],
                                  sem_s[m]).wait()

        def scale(m):
            def grp(g, _):
                vv = lax.bitcast_convert_type(
                    eb[m][2, pl.ds(g * L, L)], jnp.float32)
                for l in range(L):
                    sv = jnp.full((L,), vv[l], jnp.float32)
                    e0 = g * L + l
                    for j in range(d // L):
                        sl = pl.ds(j * L, L)
                        msg[m][e0, sl] = msg[m][e0, sl] * sv
                return 0

            lax.fori_loop(0, k // L, grp, 0)

        load_idx(0, 0)
        wait_idx(0)
        start_gather(0)
        wait_gather(0)
        scale(0)
        start_scatter(0)
        if n_chunks > 1:
            load_idx(1, 1)
            wait_idx(1)
            start_gather(1)
        if n_chunks > 2:
            load_idx(2, 2)

        def chunk_body(c, m):
            @pl.when(c + 1 < n_chunks)
            def _():
                wait_idx((m + 1) % 3)
                start_gather((m + 1) % 3)

            wait_gather(m)
            scale(m)
            wait_scatter((m + 2) % 3)
            start_scatter(m)

            @pl.when(c + 2 < n_chunks)
            def _():
                load_idx(c + 2, (m + 2) % 3)

        n_trips = (n_chunks - 1) // 3
        rem = (n_chunks - 1) - 3 * n_trips
        if n_trips > 0:
            def trip(u, _):
                for r in range(3):
                    chunk_body(3 * u + 1 + r, (1 + r) % 3)
                return 0

            lax.fori_loop(0, n_trips, trip, 0)
        for r in range(rem):
            c = 3 * n_trips + 1 + r
            chunk_body(c, c % 3)
        wait_scatter((n_chunks - 1) % 3)

        plsc.subcore_barrier()

        for off, sz in spans:
            r0 = s * rows_tile + off
            pltpu.sync_copy(acc_sh.at[pl.ds(r0, sz)], drain_v.at[pl.ds(0, sz)])
            pltpu.sync_copy(drain_v.at[pl.ds(0, sz)], out_hbm.at[c, pl.ds(r0, sz)])

    return mp(h, packed)


def kernel(X, edge_index, edge_vals, W, b):
    h = _matmul(X, W.T, b.reshape(1, -1))
    rows = edge_index[0]
    cols = edge_index[1]
    e = rows.shape[0]
    k = 104
    e_tile = -(-e // (NW * k)) * k
    e_pad = NW * e_tile
    n_chunks = e_tile // k
    pad = e_pad - e
    if pad:
        zi = jnp.zeros((pad,), jnp.int32)
        rows = jnp.concatenate([rows, zi])
        cols = jnp.concatenate([cols, zi])
        edge_vals = jnp.concatenate([edge_vals, jnp.zeros((pad,), jnp.float32)])
    packed = jnp.stack(
        [rows.reshape(-1, k), cols.reshape(-1, k),
         lax.bitcast_convert_type(edge_vals, jnp.int32).reshape(-1, k)],
        axis=1)
    partials = _sc_mp(h, packed, e_pad, k, n_chunks)
    return _combine(partials, X.shape[0])


# speedup vs baseline: 1.6503x; 1.6503x over previous
import functools

import jax
import jax.numpy as jnp
from jax import lax
from jax.experimental import pallas as pl
from jax.experimental.pallas import tpu as pltpu
from jax.experimental.pallas import tpu_sc as plsc

NC = 2
NS = 16
NW = NC * NS
L = 16


def _mm_body(x_ref, wt_ref, b_ref, h_ref):
    h_ref[...] = (
        jnp.dot(x_ref[...], wt_ref[...], preferred_element_type=jnp.float32)
        + b_ref[...]
    )


@functools.partial(jax.jit, static_argnames=())
def _matmul(x, wt, b2d):
    n, d_in = x.shape
    d_out = wt.shape[1]
    blk = 1000 if n % 1000 == 0 else n
    grid = n // blk
    return pl.pallas_call(
        _mm_body,
        grid=(grid,),
        in_specs=[
            pl.BlockSpec((blk, d_in), lambda i: (i, 0)),
            pl.BlockSpec((d_in, d_out), lambda i: (0, 0)),
            pl.BlockSpec((1, d_out), lambda i: (0, 0)),
        ],
        out_specs=pl.BlockSpec((blk, d_out), lambda i: (i, 0)),
        out_shape=jax.ShapeDtypeStruct((n, d_out), jnp.float32),
    )(x, wt, b2d)


def _comb_body(p_ref, o_ref):
    o_ref[...] = jnp.maximum(p_ref[0] + p_ref[1], 0.0)


def _combine(partials, n):
    _, _, d = partials.shape
    blk = 1000 if n % 1000 == 0 else n
    grid = n // blk
    return pl.pallas_call(
        _comb_body,
        grid=(grid,),
        in_specs=[pl.BlockSpec((2, blk, d), lambda i: (0, i, 0))],
        out_specs=pl.BlockSpec((blk, d), lambda i: (i, 0)),
        out_shape=jax.ShapeDtypeStruct((n, d), jnp.float32),
    )(partials)


def _sc_mp(h, packed, e, k, n_chunks):
    n, d = h.shape
    n_pad = -(-n // (NS * 8)) * (NS * 8)
    rows_tile = n_pad // NS
    dk = 32
    spans = []
    off = 0
    while off < rows_tile:
        spans.append((off, min(dk, rows_tile - off)))
        off += dk

    mesh = plsc.VectorSubcoreMesh(
        core_axis_name="c", subcore_axis_name="s",
        num_cores=NC, num_subcores=NS)

    @functools.partial(
        pl.kernel,
        out_type=jax.ShapeDtypeStruct((NC, n_pad, d), jnp.float32),
        mesh=mesh,
        scratch_types=[
            [pltpu.VMEM((3, k), jnp.int32) for _ in range(3)],
            [pltpu.VMEM((k, d), jnp.float32) for _ in range(3)],
            pltpu.VMEM((dk, d), jnp.float32),
            pltpu.VMEM_SHARED((n_pad, d), jnp.float32),
            [pltpu.SemaphoreType.DMA for _ in range(3)],
            [pltpu.SemaphoreType.DMA for _ in range(3)],
            [pltpu.SemaphoreType.DMA for _ in range(3)],
        ],
    )
    def mp(h_hbm, pk_hbm, out_hbm,
           eb, msg, drain_v, acc_sh, sem_i, sem_g, sem_s):
        c = lax.axis_index("c")
        s = lax.axis_index("s")
        wid = c * NS + s

        def zrow(r, _):
            for j in range(d // L):
                drain_v[r, pl.ds(j * L, L)] = jnp.zeros((L,), jnp.float32)
            return 0

        lax.fori_loop(0, dk, zrow, 0)

        for off, sz in spans:
            pltpu.sync_copy(
                drain_v.at[pl.ds(0, sz)],
                acc_sh.at[pl.ds(s * rows_tile + off, sz)])
        plsc.subcore_barrier()

        chunk0 = wid * n_chunks

        def load_idx(ci, m):
            pltpu.async_copy(pk_hbm.at[chunk0 + ci], eb[m], sem_i[m])

        def wait_idx(m):
            pltpu.make_async_copy(pk_hbm.at[0], eb[m], sem_i[m]).wait()

        def start_gather(m):
            pltpu.async_copy(h_hbm.at[eb[m].at[1]], msg[m], sem_g[m])

        def wait_gather(m):
            pltpu.make_async_copy(h_hbm.at[pl.ds(0, k)], msg[m],
                                  sem_g[m]).wait()

        def start_scatter(m):
            pltpu.async_copy(msg[m], acc_sh.at[eb[m].at[0]], sem_s[m],
                             add=True)

        def wait_scatter(m):
            pltpu.make_async_copy(msg[m], acc_sh.at[pl.ds(0, k)],
                                  sem_s[m]).wait()

        def scale(m):
            def grp(g, _):
                vv = lax.bitcast_convert_type(
                    eb[m][2, pl.ds(g * L, L)], jnp.float32)
                for l in range(L):
                    sv = jnp.full((L,), vv[l], jnp.float32)
                    e0 = g * L + l
                    for j in range(d // L):
                        sl = pl.ds(j * L, L)
                        msg[m][e0, sl] = msg[m][e0, sl] * sv
                return 0

            lax.fori_loop(0, k // L, grp, 0)

        load_idx(0, 0)
        wait_idx(0)
        start_gather(0)
        wait_gather(0)
        scale(0)
        start_scatter(0)
        if n_chunks > 1:
            load_idx(1, 1)
            wait_idx(1)
            start_gather(1)
        if n_chunks > 2:
            load_idx(2, 2)

        def chunk_body(c, m):
            @pl.when(c + 1 < n_chunks)
            def _():
                wait_idx((m + 1) % 3)
                start_gather((m + 1) % 3)

            wait_gather(m)
            scale(m)
            wait_scatter((m + 2) % 3)
            start_scatter(m)

            @pl.when(c + 2 < n_chunks)
            def _():
                load_idx(c + 2, (m + 2) % 3)

        n_trips = (n_chunks - 1) // 3
        rem = (n_chunks - 1) - 3 * n_trips
        if n_trips > 0:
            def trip(u, _):
                for r in range(3):
                    chunk_body(3 * u + 1 + r, (1 + r) % 3)
                return 0

            lax.fori_loop(0, n_trips, trip, 0)
        for r in range(rem):
            c = 3 * n_trips + 1 + r
            chunk_body(c, c % 3)
        wait_scatter((n_chunks - 1) % 3)

        plsc.subcore_barrier()

        for off, sz in spans:
            r0 = s * rows_tile + off
            pltpu.sync_copy(acc_sh.at[pl.ds(r0, sz)], drain_v.at[pl.ds(0, sz)])
            pltpu.sync_copy(drain_v.at[pl.ds(0, sz)], out_hbm.at[c, pl.ds(r0, sz)])

    return mp(h, packed)


def kernel(X, edge_index, edge_vals, W, b):
    h = _matmul(X, W.T, b.reshape(1, -1))
    rows = edge_index[0]
    cols = edge_index[1]
    e = rows.shape[0]
    k = 80
    e_tile = -(-e // (NW * k)) * k
    e_pad = NW * e_tile
    n_chunks = e_tile // k
    pad = e_pad - e
    if pad:
        zi = jnp.zeros((pad,), jnp.int32)
        rows = jnp.concatenate([rows, zi])
        cols = jnp.concatenate([cols, zi])
        edge_vals = jnp.concatenate([edge_vals, jnp.zeros((pad,), jnp.float32)])
    packed = jnp.stack(
        [rows.reshape(-1, k), cols.reshape(-1, k),
         lax.bitcast_convert_type(edge_vals, jnp.int32).reshape(-1, k)],
        axis=1)
    partials = _sc_mp(h, packed, e_pad, k, n_chunks)
    return _combine(partials, X.shape[0])


# speedup vs baseline: 1.6893x; 1.0237x over previous
import functools

import jax
import jax.numpy as jnp
from jax import lax
from jax.experimental import pallas as pl
from jax.experimental.pallas import tpu as pltpu
from jax.experimental.pallas import tpu_sc as plsc

NC = 2
NS = 16
NW = NC * NS
L = 16


def _mm_body(x_ref, wt_ref, b_ref, h_ref):
    h_ref[...] = (
        jnp.dot(x_ref[...], wt_ref[...], preferred_element_type=jnp.float32)
        + b_ref[...]
    )


@functools.partial(jax.jit, static_argnames=())
def _matmul(x, wt, b2d):
    n, d_in = x.shape
    d_out = wt.shape[1]
    blk = 1000 if n % 1000 == 0 else n
    grid = n // blk
    return pl.pallas_call(
        _mm_body,
        grid=(grid,),
        in_specs=[
            pl.BlockSpec((blk, d_in), lambda i: (i, 0)),
            pl.BlockSpec((d_in, d_out), lambda i: (0, 0)),
            pl.BlockSpec((1, d_out), lambda i: (0, 0)),
        ],
        out_specs=pl.BlockSpec((blk, d_out), lambda i: (i, 0)),
        out_shape=jax.ShapeDtypeStruct((n, d_out), jnp.float32),
    )(x, wt, b2d)


def _comb_body(p_ref, o_ref):
    o_ref[...] = jnp.maximum(p_ref[0] + p_ref[1], 0.0)


def _combine(partials, n):
    _, _, d = partials.shape
    blk = 1000 if n % 1000 == 0 else n
    grid = n // blk
    return pl.pallas_call(
        _comb_body,
        grid=(grid,),
        in_specs=[pl.BlockSpec((2, blk, d), lambda i: (0, i, 0))],
        out_specs=pl.BlockSpec((blk, d), lambda i: (i, 0)),
        out_shape=jax.ShapeDtypeStruct((n, d), jnp.float32),
    )(partials)


def _sc_mp(h, packed, e, k, n_chunks):
    n, d = h.shape
    n_pad = -(-n // (NS * 8)) * (NS * 8)
    rows_tile = n_pad // NS
    spans = []
    off = 0
    while off < rows_tile:
        spans.append((off, min(k, rows_tile - off)))
        off += k

    mesh = plsc.VectorSubcoreMesh(
        core_axis_name="c", subcore_axis_name="s",
        num_cores=NC, num_subcores=NS)

    @functools.partial(
        pl.kernel,
        out_type=jax.ShapeDtypeStruct((NC, n_pad, d), jnp.float32),
        mesh=mesh,
        scratch_types=[
            [pltpu.VMEM((3, k), jnp.int32) for _ in range(3)],
            [pltpu.VMEM((k, d), jnp.float32) for _ in range(3)],
            pltpu.VMEM_SHARED((n_pad, d), jnp.float32),
            [pltpu.SemaphoreType.DMA for _ in range(3)],
            [pltpu.SemaphoreType.DMA for _ in range(3)],
            [pltpu.SemaphoreType.DMA for _ in range(3)],
        ],
    )
    def mp(h_hbm, pk_hbm, out_hbm,
           eb, msg, acc_sh, sem_i, sem_g, sem_s):
        c = lax.axis_index("c")
        s = lax.axis_index("s")
        wid = c * NS + s

        def zrow(r, _):
            for j in range(d // L):
                msg[2][r, pl.ds(j * L, L)] = jnp.zeros((L,), jnp.float32)
            return 0

        lax.fori_loop(0, k, zrow, 0)

        for off, sz in spans:
            pltpu.async_copy(
                msg[2].at[pl.ds(0, sz)],
                acc_sh.at[pl.ds(s * rows_tile + off, sz)], sem_i[2])
        for off, sz in spans:
            pltpu.make_async_copy(
                msg[2].at[pl.ds(0, sz)],
                acc_sh.at[pl.ds(0, sz)], sem_i[2]).wait()
        plsc.subcore_barrier()

        chunk0 = wid * n_chunks

        def load_idx(ci, m):
            pltpu.async_copy(pk_hbm.at[chunk0 + ci], eb[m], sem_i[m])

        def wait_idx(m):
            pltpu.make_async_copy(pk_hbm.at[0], eb[m], sem_i[m]).wait()

        def start_gather(m):
            pltpu.async_copy(h_hbm.at[eb[m].at[1]], msg[m], sem_g[m])

        def wait_gather(m):
            pltpu.make_async_copy(h_hbm.at[pl.ds(0, k)], msg[m],
                                  sem_g[m]).wait()

        def start_scatter(m):
            pltpu.async_copy(msg[m], acc_sh.at[eb[m].at[0]], sem_s[m],
                             add=True)

        def wait_scatter(m):
            pltpu.make_async_copy(msg[m], acc_sh.at[eb[m].at[0]],
                                  sem_s[m]).wait()

        def scale(m):
            def grp(g, _):
                vv = lax.bitcast_convert_type(
                    eb[m][2, pl.ds(g * L, L)], jnp.float32)
                for l in range(L):
                    sv = jnp.full((L,), vv[l], jnp.float32)
                    e0 = g * L + l
                    for j in range(d // L):
                        sl = pl.ds(j * L, L)
                        msg[m][e0, sl] = msg[m][e0, sl] * sv
                return 0

            lax.fori_loop(0, k // L, grp, 0)

        load_idx(0, 0)
        wait_idx(0)
        start_gather(0)
        wait_gather(0)
        scale(0)
        start_scatter(0)
        if n_chunks > 1:
            load_idx(1, 1)
            wait_idx(1)
            start_gather(1)
        if n_chunks > 2:
            load_idx(2, 2)

        def chunk_body(c, m):
            @pl.when(c + 1 < n_chunks)
            def _():
                wait_idx((m + 1) % 3)
                start_gather((m + 1) % 3)

            wait_gather(m)
            scale(m)
            wait_scatter((m + 2) % 3)
            start_scatter(m)

            @pl.when(c + 2 < n_chunks)
            def _():
                load_idx(c + 2, (m + 2) % 3)

        n_trips = (n_chunks - 1) // 3
        rem = (n_chunks - 1) - 3 * n_trips
        if n_trips > 0:
            def trip(u, _):
                for r in range(3):
                    chunk_body(3 * u + 1 + r, (1 + r) % 3)
                return 0

            lax.fori_loop(0, n_trips, trip, 0)
        for r in range(rem):
            c = 3 * n_trips + 1 + r
            chunk_body(c, c % 3)
        wait_scatter((n_chunks - 1) % 3)

        plsc.subcore_barrier()

        prev = [None, None]
        for i, (off, sz) in enumerate(spans):
            m = i % 2
            r0 = s * rows_tile + off
            if prev[m] is not None:
                pltpu.make_async_copy(
                    msg[m].at[pl.ds(0, prev[m])],
                    out_hbm.at[c, pl.ds(0, prev[m])], sem_g[m]).wait()
            pltpu.sync_copy(acc_sh.at[pl.ds(r0, sz)], msg[m].at[pl.ds(0, sz)])
            pltpu.async_copy(
                msg[m].at[pl.ds(0, sz)], out_hbm.at[c, pl.ds(r0, sz)],
                sem_g[m])
            prev[m] = sz
        for m in (0, 1):
            if prev[m] is not None:
                pltpu.make_async_copy(
                    msg[m].at[pl.ds(0, prev[m])],
                    out_hbm.at[c, pl.ds(0, prev[m])], sem_g[m]).wait()

    return mp(h, packed)


def kernel(X, edge_index, edge_vals, W, b):
    h = _matmul(X, W.T, b.reshape(1, -1))
    rows = edge_index[0]
    cols = edge_index[1]
    e = rows.shape[0]
    k = 80
    e_tile = -(-e // (NW * k)) * k
    e_pad = NW * e_tile
    n_chunks = e_tile // k
    pad = e_pad - e
    if pad:
        zi = jnp.zeros((pad,), jnp.int32)
        rows = jnp.concatenate([rows, zi])
        cols = jnp.concatenate([cols, zi])
        edge_vals = jnp.concatenate([edge_vals, jnp.zeros((pad,), jnp.float32)])
    packed = jnp.stack(
        [rows.reshape(-1, k), cols.reshape(-1, k),
         lax.bitcast_convert_type(edge_vals, jnp.int32).reshape(-1, k)],
        axis=1)
    partials = _sc_mp(h, packed, e_pad, k, n_chunks)
    return _combine(partials, X.shape[0])
